# Initial kernel scaffold; baseline (speedup 1.0000x reference)
import jax
import jax.numpy as jnp
from jax.experimental import pallas as pl


def kernel(x, edge_index, W1, b1, W2, b2):
    raise NotImplementedError("write your pallas kernel here")


# speedup vs baseline: 39.5754x; 39.5754x over previous
import functools

import jax
import jax.numpy as jnp
from jax import lax
from jax.experimental import pallas as pl
from jax.experimental.pallas import tpu as pltpu
from jax.experimental.pallas import tpu_sc as plsc

_NC = 2
_NS = 16
_LANES = 16
_CHUNK = 128


def _sc_mesh():
    return plsc.VectorSubcoreMesh(
        core_axis_name="c", subcore_axis_name="s",
        num_cores=_NC, num_subcores=_NS)


_ZBLK = 64


def _sc_degree(dst_w, *, n_pad, cw):
    R = n_pad // _NS

    @functools.partial(
        pl.kernel,
        out_type=jax.ShapeDtypeStruct((_NC, n_pad, _LANES), jnp.float32),
        mesh=_sc_mesh(),
        compiler_params=pltpu.CompilerParams(use_tc_tiling_on_sc=False),
        scratch_types=[
            pltpu.VMEM_SHARED((n_pad, _LANES), jnp.float32),
            pltpu.VMEM((_CHUNK, _LANES), jnp.float32),
            pltpu.VMEM((cw, _CHUNK), jnp.int32),
            pltpu.VMEM((_ZBLK, _LANES), jnp.float32),
        ],
    )
    def k(dst_hbm, deg_hbm, deg_sh, ones_v, dst_v, zerov):
        c = lax.axis_index("c")
        s = lax.axis_index("s")
        w = c * _NS + s
        row0 = s * R

        pltpu.sync_copy(dst_hbm.at[w], dst_v)

        fzero = jnp.zeros((_LANES,), jnp.float32)
        fone = jnp.full((_LANES,), 1.0, jnp.float32)

        def zfill_body(i, _):
            zerov[i] = fzero
            return 0
        lax.fori_loop(0, _ZBLK, zfill_body, 0)

        def ones_body(i, _):
            ones_v[i] = fone
            return 0
        lax.fori_loop(0, _CHUNK, ones_body, 0)

        def zero_body(i, _):
            pltpu.sync_copy(zerov, deg_sh.at[pl.ds(row0 + i * _ZBLK, _ZBLK)])
            return 0
        lax.fori_loop(0, R // _ZBLK, zero_body, 0)
        plsc.subcore_barrier()

        def deg_body(j, _):
            pltpu.sync_copy(ones_v, deg_sh.at[dst_v.at[j]], add=True)
            return 0
        lax.fori_loop(0, cw, deg_body, 0)
        plsc.subcore_barrier()

        pltpu.sync_copy(deg_sh.at[pl.ds(row0, R)], deg_hbm.at[c, pl.ds(row0, R)])

    return k(dst_w)


def _sc_aggregate(hp, src_w, dst_w, *, n_pad, cw):
    R = n_pad // _NS

    @functools.partial(
        pl.kernel,
        out_type=jax.ShapeDtypeStruct((_NC, n_pad, _LANES), jnp.float32),
        mesh=_sc_mesh(),
        compiler_params=pltpu.CompilerParams(use_tc_tiling_on_sc=False),
        scratch_types=[
            pltpu.VMEM_SHARED((n_pad, _LANES), jnp.float32),
            pltpu.VMEM_SHARED((n_pad, _LANES), jnp.float32),
            pltpu.VMEM((cw, _CHUNK), jnp.int32),
            pltpu.VMEM((cw, _CHUNK), jnp.int32),
            pltpu.VMEM((_CHUNK, _LANES), jnp.float32),
            pltpu.VMEM((_ZBLK, _LANES), jnp.float32),
            pltpu.SemaphoreType.DMA,
        ],
    )
    def k(hp_hbm, src_hbm, dst_hbm, acc_hbm,
          hp_sh, acc_sh, src_v, dst_v, rows_v, zerov, sem):
        c = lax.axis_index("c")
        s = lax.axis_index("s")
        w = c * _NS + s
        row0 = s * R

        pltpu.sync_copy(src_hbm.at[w], src_v)
        pltpu.sync_copy(dst_hbm.at[w], dst_v)
        pltpu.sync_copy(hp_hbm.at[pl.ds(row0, R)], hp_sh.at[pl.ds(row0, R)])

        fzero = jnp.zeros((_LANES,), jnp.float32)

        def zfill_body(i, _):
            zerov[i] = fzero
            return 0
        lax.fori_loop(0, _ZBLK, zfill_body, 0)

        def zero_body(i, _):
            pltpu.sync_copy(zerov, acc_sh.at[pl.ds(row0 + i * _ZBLK, _ZBLK)])
            return 0
        lax.fori_loop(0, R // _ZBLK, zero_body, 0)
        plsc.subcore_barrier()

        def edge_body(j, _):
            pltpu.async_copy(hp_sh.at[src_v.at[j]], rows_v, sem).wait()
            pltpu.sync_copy(rows_v, acc_sh.at[dst_v.at[j]], add=True)
            return 0
        lax.fori_loop(0, cw, edge_body, 0)
        plsc.subcore_barrier()

        pltpu.sync_copy(acc_sh.at[pl.ds(row0, R)], acc_hbm.at[c, pl.ds(row0, R)])

    return k(hp, src_w, dst_w)


def _tc_linear(x_pad, w1, b1, *, n, n_pad, h):
    blk = 256
    grid = n_pad // blk

    def body(x_ref, w_ref, b_ref, o_ref):
        i = pl.program_id(0)
        acc = jnp.dot(x_ref[...], w_ref[...],
                      preferred_element_type=jnp.float32) + b_ref[...]
        rows = i * blk + lax.broadcasted_iota(jnp.int32, (blk, h), 0)
        o_ref[...] = jnp.where(rows < n, acc, 0.0)

    d = x_pad.shape[1]
    return pl.pallas_call(
        body,
        grid=(grid,),
        in_specs=[
            pl.BlockSpec((blk, d), lambda i: (i, 0)),
            pl.BlockSpec((d, h), lambda i: (0, 0)),
            pl.BlockSpec((1, h), lambda i: (0, 0)),
        ],
        out_specs=pl.BlockSpec((blk, h), lambda i: (i, 0)),
        out_shape=jax.ShapeDtypeStruct((n_pad, h), jnp.float32),
    )(x_pad, w1, b1.reshape(1, h))


def _tc_scale(deg_pair, h_pad, *, n_pad, h):
    blk = 256
    grid = n_pad // blk

    def body(d0_ref, d1_ref, h_ref, hp_ref, dis_ref):
        deg = d0_ref[...] + d1_ref[...] + 1.0
        dis = lax.rsqrt(deg)
        dis_ref[...] = dis
        hp_ref[...] = dis * h_ref[...]

    specs = pl.BlockSpec((blk, h), lambda i: (i, 0))
    return pl.pallas_call(
        body,
        grid=(grid,),
        in_specs=[specs, specs, specs],
        out_specs=[specs, specs],
        out_shape=[jax.ShapeDtypeStruct((n_pad, h), jnp.float32),
                   jax.ShapeDtypeStruct((n_pad, h), jnp.float32)],
    )(deg_pair[0], deg_pair[1], h_pad)


def _tc_head(y0, y1, hp, dis, w2, b2, *, n_pad, h, c):
    blk = 256
    grid = n_pad // blk

    def body(a_ref, b_ref, hp_ref, dis_ref, w_ref, bias_ref, o_ref):
        z = dis_ref[...] * (a_ref[...] + b_ref[...] + hp_ref[...])
        z = jnp.maximum(z, 0.0)
        logits = jnp.dot(z, w_ref[...],
                         preferred_element_type=jnp.float32) + bias_ref[...]
        m = jnp.max(logits, axis=1, keepdims=True)
        e = jnp.exp(logits - m)
        o_ref[...] = e / jnp.sum(e, axis=1, keepdims=True)

    rows = pl.BlockSpec((blk, h), lambda i: (i, 0))
    return pl.pallas_call(
        body,
        grid=(grid,),
        in_specs=[
            rows, rows, rows, rows,
            pl.BlockSpec((h, c), lambda i: (0, 0)),
            pl.BlockSpec((1, c), lambda i: (0, 0)),
        ],
        out_specs=pl.BlockSpec((blk, c), lambda i: (i, 0)),
        out_shape=jax.ShapeDtypeStruct((n_pad, c), jnp.float32),
    )(y0, y1, hp, dis, w2, b2.reshape(1, c))


def kernel(x, edge_index, W1, b1, W2, b2):
    n, d = x.shape
    h = W1.shape[1]
    c = W2.shape[1]
    e = edge_index.shape[1]

    n_pad = -(-(n + 64) // 256) * 256
    junk = n_pad - n
    epw = _NC * _NS * _CHUNK
    e_pad = -(-e // epw) * epw
    cw = e_pad // epw

    x_pad = jnp.pad(x, ((0, n_pad - n), (0, 0)))
    h_pad = _tc_linear(x_pad, W1, b1, n=n, n_pad=n_pad, h=h)

    pad_cnt = e_pad - e
    pad_idx = n + jnp.arange(pad_cnt, dtype=jnp.int32) % junk
    src = jnp.concatenate([edge_index[0], pad_idx])
    dst = jnp.concatenate([edge_index[1], pad_idx])
    src_w = src.reshape(_NC * _NS, cw, _CHUNK)
    dst_w = dst.reshape(_NC * _NS, cw, _CHUNK)

    deg_pair = _sc_degree(dst_w, n_pad=n_pad, cw=cw)
    hp, dis = _tc_scale(deg_pair, h_pad, n_pad=n_pad, h=h)
    acc_pair = _sc_aggregate(hp, src_w, dst_w, n_pad=n_pad, cw=cw)
    out = _tc_head(acc_pair[0], acc_pair[1], hp, dis, W2, b2,
                   n_pad=n_pad, h=h, c=c)
    return out[:n]


# speedup vs baseline: 51.6002x; 1.3038x over previous
import functools

import jax
import jax.numpy as jnp
from jax import lax
from jax.experimental import pallas as pl
from jax.experimental.pallas import tpu as pltpu
from jax.experimental.pallas import tpu_sc as plsc

_NC = 2
_NS = 16
_LANES = 16
_CHUNK = 128
_NBUF = 4
_ZBLK = 64


def _sc_mesh():
    return plsc.VectorSubcoreMesh(
        core_axis_name="c", subcore_axis_name="s",
        num_cores=_NC, num_subcores=_NS)


_SC_PARAMS = pltpu.CompilerParams(use_tc_tiling_on_sc=False,
                                  needs_layout_passes=False)


def _sc_degree(dst_w, *, n_pad, cw):
    R = n_pad // _NS

    @functools.partial(
        pl.kernel,
        out_type=jax.ShapeDtypeStruct((_NC, n_pad, _LANES), jnp.float32),
        mesh=_sc_mesh(),
        compiler_params=_SC_PARAMS,
        scratch_types=[
            pltpu.VMEM_SHARED((n_pad, _LANES), jnp.float32),
            pltpu.VMEM((_CHUNK, _LANES), jnp.float32),
            pltpu.VMEM((cw, _CHUNK), jnp.int32),
            pltpu.VMEM((_ZBLK, _LANES), jnp.float32),
            pltpu.SemaphoreType.DMA,
        ],
    )
    def k(dst_hbm, deg_hbm, deg_sh, ones_v, dst_v, zerov, sem):
        c = lax.axis_index("c")
        s = lax.axis_index("s")
        w = c * _NS + s
        row0 = s * R

        pltpu.sync_copy(dst_hbm.at[w], dst_v)

        fzero = jnp.zeros((_LANES,), jnp.float32)
        fone = jnp.full((_LANES,), 1.0, jnp.float32)

        def zfill_body(i, _):
            zerov[i] = fzero
            return 0
        lax.fori_loop(0, _ZBLK, zfill_body, 0)

        def ones_body(i, _):
            ones_v[i] = fone
            return 0
        lax.fori_loop(0, _CHUNK, ones_body, 0)

        def zero_body(i, _):
            pltpu.sync_copy(zerov, deg_sh.at[pl.ds(row0 + i * _ZBLK, _ZBLK)])
            return 0
        lax.fori_loop(0, R // _ZBLK, zero_body, 0)
        plsc.subcore_barrier()

        def deg_body(g, _):
            descs = [
                pltpu.async_copy(
                    ones_v, deg_sh.at[dst_v.at[_NBUF * g + b]], sem, add=True)
                for b in range(_NBUF)
            ]
            for d in descs:
                d.wait()
            return 0
        lax.fori_loop(0, cw // _NBUF, deg_body, 0)
        plsc.subcore_barrier()

        pltpu.sync_copy(deg_sh.at[pl.ds(row0, R)], deg_hbm.at[c, pl.ds(row0, R)])

    return k(dst_w)


def _sc_aggregate(h_pad, deg_pair, src_w, dst_w, *, n_pad, cw):
    R = n_pad // _NS

    @functools.partial(
        pl.kernel,
        out_type=jax.ShapeDtypeStruct((_NC, n_pad, _LANES), jnp.float32),
        mesh=_sc_mesh(),
        compiler_params=_SC_PARAMS,
        scratch_types=[
            pltpu.VMEM_SHARED((n_pad, _LANES), jnp.float32),
            pltpu.VMEM_SHARED((n_pad, _LANES), jnp.float32),
            pltpu.VMEM((cw, _CHUNK), jnp.int32),
            pltpu.VMEM((cw, _CHUNK), jnp.int32),
            [pltpu.VMEM((_CHUNK, _LANES), jnp.float32)
             for _ in range(_NBUF)],
            pltpu.VMEM((R, _LANES), jnp.float32),
            pltpu.VMEM((R, _LANES), jnp.float32),
            pltpu.VMEM((R, _LANES), jnp.float32),
            pltpu.VMEM((R, _LANES), jnp.float32),
            pltpu.VMEM((R, _LANES), jnp.float32),
            pltpu.VMEM((_ZBLK, _LANES), jnp.float32),
            pltpu.SemaphoreType.DMA,
        ],
    )
    def k(h_hbm, deg_hbm, src_hbm, dst_hbm, y_hbm,
          hp_sh, acc_sh, src_v, dst_v, rows,
          hv, d0v, d1v, hpv, disv, zerov, sem):
        c = lax.axis_index("c")
        s = lax.axis_index("s")
        w = c * _NS + s
        row0 = s * R

        pltpu.sync_copy(src_hbm.at[w], src_v)
        pltpu.sync_copy(dst_hbm.at[w], dst_v)
        pltpu.sync_copy(h_hbm.at[pl.ds(row0, R)], hv)
        pltpu.sync_copy(deg_hbm.at[0, pl.ds(row0, R)], d0v)
        pltpu.sync_copy(deg_hbm.at[1, pl.ds(row0, R)], d1v)

        fzero = jnp.zeros((_LANES,), jnp.float32)
        fone = jnp.full((_LANES,), 1.0, jnp.float32)
        half = jnp.full((_LANES,), 0.5, jnp.float32)
        three_half = jnp.full((_LANES,), 1.5, jnp.float32)
        magic = jnp.full((_LANES,), 0x5F3759DF, jnp.int32)
        one_i = jnp.full((_LANES,), 1, jnp.int32)

        def zfill_body(i, _):
            zerov[i] = fzero
            return 0
        lax.fori_loop(0, _ZBLK, zfill_body, 0)

        def zero_body(i, _):
            pltpu.sync_copy(zerov, acc_sh.at[pl.ds(row0 + i * _ZBLK, _ZBLK)])
            return 0
        lax.fori_loop(0, R // _ZBLK, zero_body, 0)

        def rs_body(i, _):
            d = d0v[i] + d1v[i] + fone
            bits = plsc.bitcast(d, jnp.int32)
            y = plsc.bitcast(
                magic - lax.shift_right_arithmetic(bits, one_i), jnp.float32)
            hd = half * d
            y = y * (three_half - hd * y * y)
            y = y * (three_half - hd * y * y)
            y = y * (three_half - hd * y * y)
            disv[i] = y
            hpv[i] = hv[i] * y
            return 0
        lax.fori_loop(0, R, rs_body, 0)
        pltpu.sync_copy(hpv, hp_sh.at[pl.ds(row0, R)])
        plsc.subcore_barrier()

        def edge_body(g, _):
            descs = [
                pltpu.async_copy(
                    hp_sh.at[src_v.at[_NBUF * g + b]], rows[b], sem)
                for b in range(_NBUF)
            ]
            for b in range(_NBUF):
                descs[b].wait()
                pltpu.sync_copy(
                    rows[b], acc_sh.at[dst_v.at[_NBUF * g + b]], add=True)
            return 0
        lax.fori_loop(0, cw // _NBUF, edge_body, 0)
        plsc.subcore_barrier()

        pltpu.sync_copy(acc_sh.at[pl.ds(row0, R)], d0v)

        def y_body(i, _):
            d1v[i] = disv[i] * (d0v[i] + half * hpv[i])
            return 0
        lax.fori_loop(0, R, y_body, 0)
        pltpu.sync_copy(d1v, y_hbm.at[c, pl.ds(row0, R)])

    return k(h_pad, deg_pair, src_w, dst_w)


def _tc_linear(x_pad, w1, b1, *, n, n_pad, h):
    blk = 256
    grid = n_pad // blk

    def body(x_ref, w_ref, b_ref, o_ref):
        i = pl.program_id(0)
        acc = jnp.dot(x_ref[...], w_ref[...],
                      preferred_element_type=jnp.float32) + b_ref[...]
        rows = i * blk + lax.broadcasted_iota(jnp.int32, (blk, h), 0)
        o_ref[...] = jnp.where(rows < n, acc, 0.0)

    d = x_pad.shape[1]
    return pl.pallas_call(
        body,
        grid=(grid,),
        in_specs=[
            pl.BlockSpec((blk, d), lambda i: (i, 0)),
            pl.BlockSpec((d, h), lambda i: (0, 0)),
            pl.BlockSpec((1, h), lambda i: (0, 0)),
        ],
        out_specs=pl.BlockSpec((blk, h), lambda i: (i, 0)),
        out_shape=jax.ShapeDtypeStruct((n_pad, h), jnp.float32),
    )(x_pad, w1, b1.reshape(1, h))


def _tc_head(y0, y1, w2, b2, *, n_pad, h, c):
    blk = 256
    grid = n_pad // blk

    def body(a_ref, b_ref, w_ref, bias_ref, o_ref):
        z = jnp.maximum(a_ref[...] + b_ref[...], 0.0)
        logits = jnp.dot(z, w_ref[...],
                         preferred_element_type=jnp.float32) + bias_ref[...]
        m = jnp.max(logits, axis=1, keepdims=True)
        e = jnp.exp(logits - m)
        o_ref[...] = e / jnp.sum(e, axis=1, keepdims=True)

    rows = pl.BlockSpec((blk, h), lambda i: (i, 0))
    return pl.pallas_call(
        body,
        grid=(grid,),
        in_specs=[
            rows, rows,
            pl.BlockSpec((h, c), lambda i: (0, 0)),
            pl.BlockSpec((1, c), lambda i: (0, 0)),
        ],
        out_specs=pl.BlockSpec((blk, c), lambda i: (i, 0)),
        out_shape=jax.ShapeDtypeStruct((n_pad, c), jnp.float32),
    )(y0, y1, w2, b2.reshape(1, c))


def kernel(x, edge_index, W1, b1, W2, b2):
    n, d = x.shape
    h = W1.shape[1]
    c = W2.shape[1]
    e = edge_index.shape[1]

    n_pad = -(-(n + 64) // 256) * 256
    junk = n_pad - n
    epw = _NC * _NS * _CHUNK * _NBUF
    e_pad = -(-e // epw) * epw
    cw = e_pad // (_NC * _NS * _CHUNK)

    x_pad = jnp.pad(x, ((0, n_pad - n), (0, 0)))
    h_pad = _tc_linear(x_pad, W1, b1, n=n, n_pad=n_pad, h=h)

    pad_cnt = e_pad - e
    pad_idx = n + jnp.arange(pad_cnt, dtype=jnp.int32) % junk
    src = jnp.concatenate([edge_index[0], pad_idx])
    dst = jnp.concatenate([edge_index[1], pad_idx])
    src_w = src.reshape(_NC * _NS, cw, _CHUNK)
    dst_w = dst.reshape(_NC * _NS, cw, _CHUNK)

    deg_pair = _sc_degree(dst_w, n_pad=n_pad, cw=cw)
    y = _sc_aggregate(h_pad, deg_pair, src_w, dst_w, n_pad=n_pad, cw=cw)
    out = _tc_head(y[0], y[1], W2, b2, n_pad=n_pad, h=h, c=c)
    return out[:n]
